# baseline (speedup 1.0000x reference)
import functools

import jax
import jax.numpy as jnp
from jax import lax
from jax.experimental import pallas as pl
from jax.experimental.pallas import tpu as pltpu
from jax.experimental.pallas import tpu_sc as plsc

_B = 16384
_GD = 32
_MD = 128
_NC, _NS = 2, 16
_NW = _NC * _NS
_BPW = _B // _NW
_CH = 128
_NCH = _BPW // _CH
_MHALF = _NCH // 2

_MESH = plsc.VectorSubcoreMesh(core_axis_name="c", subcore_axis_name="s")


def _worker_id():
    return lax.axis_index("s") * _NC + lax.axis_index("c")


def _gather2_body(aidx, bidx, atab, btab, a_out, b_out,
                  aidx_v, bidx_v, a_v, b_v, sem):
    wid = _worker_id()
    base = wid * _BPW
    pltpu.sync_copy(aidx.at[pl.ds(base, _BPW)], aidx_v)
    pltpu.sync_copy(bidx.at[pl.ds(base, _BPW)], bidx_v)
    half = _MHALF * _CH
    for h in range(2):
        cps = []
        for k in range(_MHALF):
            off = h * half + k * _CH
            cps.append(pltpu.async_copy(
                atab.at[aidx_v.at[pl.ds(off, _CH)]],
                a_v.at[pl.ds(k * _CH, _CH)], sem))
            cps.append(pltpu.async_copy(
                btab.at[bidx_v.at[pl.ds(off, _CH)]],
                b_v.at[pl.ds(k * _CH, _CH)], sem))
        for cp in cps:
            cp.wait()
        pltpu.sync_copy(a_v, a_out.at[pl.ds(base + h * half, half)])
        pltpu.sync_copy(b_v, b_out.at[pl.ds(base + h * half, half)])


_gather2 = functools.partial(
    pl.kernel,
    out_type=(
        jax.ShapeDtypeStruct((_B, _MD), jnp.float32),
        jax.ShapeDtypeStruct((_B, _MD), jnp.float32),
    ),
    mesh=_MESH,
    scratch_types=[
        pltpu.VMEM((_BPW,), jnp.int32),
        pltpu.VMEM((_BPW,), jnp.int32),
        pltpu.VMEM((_MHALF * _CH, _MD), jnp.float32),
        pltpu.VMEM((_MHALF * _CH, _MD), jnp.float32),
        pltpu.SemaphoreType.DMA,
    ],
)(_gather2_body)


def _quarter_select(sup, q):
    out = jnp.where(q == 0, sup[:, 0 * _GD:1 * _GD], 0.0)
    out = out + jnp.where(q == 1, sup[:, 1 * _GD:2 * _GD], 0.0)
    out = out + jnp.where(q == 2, sup[:, 2 * _GD:3 * _GD], 0.0)
    out = out + jnp.where(q == 3, sup[:, 3 * _GD:4 * _GD], 0.0)
    return out


def _tc_mlp_body(ugs, igs, qu, qi, um, im,
                 w1u, w1i, b1, w2, b2, w3, b3, wf, bf, out):
    h = jnp.dot(um[...], w1u[...], preferred_element_type=jnp.float32)
    h = h + jnp.dot(im[...], w1i[...], preferred_element_type=jnp.float32)
    h = jnp.maximum(h + b1[...], 0.0)
    h = jnp.maximum(jnp.dot(h, w2[...], preferred_element_type=jnp.float32) + b2[...], 0.0)
    h = jnp.maximum(jnp.dot(h, w3[...], preferred_element_type=jnp.float32) + b3[...], 0.0)
    g = _quarter_select(ugs[...], qu[...].reshape(-1, 1)) * \
        _quarter_select(igs[...], qi[...].reshape(-1, 1))
    r = jnp.sum(g * wf[:, :_GD], axis=1) + jnp.sum(h * wf[:, _GD:], axis=1)
    out[...] = r + bf[0, 0]


def _tc_mlp(ugs, igs, qu, qi, um, im, w1u, w1i, b1, w2, b2, w3, b3, wf, bf):
    blk = 2048
    grid = (_B // blk,)
    fixed = lambda shape: pl.BlockSpec(shape, lambda i: (0,) * len(shape))
    return pl.pallas_call(
        _tc_mlp_body,
        grid=grid,
        in_specs=[
            pl.BlockSpec((blk, _MD), lambda i: (i, 0)),
            pl.BlockSpec((blk, _MD), lambda i: (i, 0)),
            pl.BlockSpec((blk,), lambda i: (i,)),
            pl.BlockSpec((blk,), lambda i: (i,)),
            pl.BlockSpec((blk, _MD), lambda i: (i, 0)),
            pl.BlockSpec((blk, _MD), lambda i: (i, 0)),
            fixed((_MD, _MD)),
            fixed((_MD, _MD)),
            fixed((1, _MD)),
            fixed((_MD, 64)),
            fixed((1, 64)),
            fixed((64, _GD)),
            fixed((1, _GD)),
            fixed((1, 2 * _GD)),
            fixed((1, 1)),
        ],
        out_specs=pl.BlockSpec((blk,), lambda i: (i,)),
        out_shape=jax.ShapeDtypeStruct((_B,), jnp.float32),
    )(ugs, igs, qu, qi, um, im, w1u, w1i, b1, w2, b2, w3, b3, wf, bf)


def kernel(user_indices, item_indices, user_gmf_table, item_gmf_table,
           user_mlp_table, item_mlp_table, W1, b1, W2, b2, W3, b3, Wf, bf):
    ugmf4 = user_gmf_table.reshape(-1, 4 * _GD)
    igmf4 = item_gmf_table.reshape(-1, 4 * _GD)
    su = user_indices // 4
    qu = user_indices % 4
    si = item_indices // 4
    qi = item_indices % 4
    ugs, igs = _gather2(su, si, ugmf4, igmf4)
    um, im = _gather2(user_indices, item_indices, user_mlp_table, item_mlp_table)
    w1u = W1[:, :_MD].T
    w1i = W1[:, _MD:].T
    return _tc_mlp(ugs, igs, qu, qi, um, im, w1u, w1i, b1.reshape(1, _MD),
                   W2.T, b2.reshape(1, 64), W3.T, b3.reshape(1, _GD),
                   Wf, bf.reshape(1, 1))


# speedup vs baseline: 5.7840x; 5.7840x over previous
import functools

import jax
import jax.numpy as jnp
from jax import lax
from jax.experimental import pallas as pl
from jax.experimental.pallas import tpu as pltpu
from jax.experimental.pallas import tpu_sc as plsc

_B = 16384
_GD = 32
_MD = 128
_NC, _NS = 2, 16
_NW = _NC * _NS
_BPW = _B // _NW
_CH = 128
_NCH = _BPW // _CH
_MHALF = _NCH // 2

_MESH = plsc.VectorSubcoreMesh(core_axis_name="c", subcore_axis_name="s")


def _sc_mlp_body(uidx, iidx, umlp, imlp, um_out, im_out,
                 uidx_v, iidx_v, um_v, im_v, sem):
    wid = lax.axis_index("s") * _NC + lax.axis_index("c")
    base = wid * _BPW
    pltpu.sync_copy(uidx.at[pl.ds(base, _BPW)], uidx_v)
    pltpu.sync_copy(iidx.at[pl.ds(base, _BPW)], iidx_v)
    half = _MHALF * _CH
    for h in range(2):
        cps = []
        for k in range(_MHALF):
            off = h * half + k * _CH
            cps.append(pltpu.async_copy(
                umlp.at[uidx_v.at[pl.ds(off, _CH)]],
                um_v.at[pl.ds(k * _CH, _CH)], sem))
            cps.append(pltpu.async_copy(
                imlp.at[iidx_v.at[pl.ds(off, _CH)]],
                im_v.at[pl.ds(k * _CH, _CH)], sem))
        for cp in cps:
            cp.wait()
        pltpu.sync_copy(um_v, um_out.at[pl.ds(base + h * half, half)])
        pltpu.sync_copy(im_v, im_out.at[pl.ds(base + h * half, half)])


_sc_mlp = functools.partial(
    pl.kernel,
    out_type=(
        jax.ShapeDtypeStruct((_B, _MD), jnp.float32),
        jax.ShapeDtypeStruct((_B, _MD), jnp.float32),
    ),
    mesh=_MESH,
    scratch_types=[
        pltpu.VMEM((_BPW,), jnp.int32),
        pltpu.VMEM((_BPW,), jnp.int32),
        pltpu.VMEM((_MHALF * _CH, _MD), jnp.float32),
        pltpu.VMEM((_MHALF * _CH, _MD), jnp.float32),
        pltpu.SemaphoreType.DMA,
    ],
)(_sc_mlp_body)


def _tc_mlp_body(ugT, igT, um, im, w1u, w1i, b1, w2, b2, w3, b3,
                 wfg, wfm, bf, out):
    h = jnp.dot(um[...], w1u[...], preferred_element_type=jnp.float32)
    h = h + jnp.dot(im[...], w1i[...], preferred_element_type=jnp.float32)
    h = jnp.maximum(h + b1[...], 0.0)
    h = jnp.maximum(jnp.dot(h, w2[...], preferred_element_type=jnp.float32) + b2[...], 0.0)
    h = jnp.maximum(jnp.dot(h, w3[...], preferred_element_type=jnp.float32) + b3[...], 0.0)
    gmf = jnp.sum(ugT[...] * igT[...] * wfg[...], axis=0)
    out[...] = gmf + jnp.sum(h * wfm[...], axis=1) + bf[0, 0]


def _tc_mlp(ugT, igT, um, im, w1u, w1i, b1, w2, b2, w3, b3, wfg, wfm, bf):
    blk = 2048
    grid = (_B // blk,)
    fixed = lambda shape: pl.BlockSpec(shape, lambda i: (0,) * len(shape))
    return pl.pallas_call(
        _tc_mlp_body,
        grid=grid,
        in_specs=[
            pl.BlockSpec((_GD, blk), lambda i: (0, i)),
            pl.BlockSpec((_GD, blk), lambda i: (0, i)),
            pl.BlockSpec((blk, _MD), lambda i: (i, 0)),
            pl.BlockSpec((blk, _MD), lambda i: (i, 0)),
            fixed((_MD, _MD)),
            fixed((_MD, _MD)),
            fixed((1, _MD)),
            fixed((_MD, 64)),
            fixed((1, 64)),
            fixed((64, _GD)),
            fixed((1, _GD)),
            fixed((_GD, 1)),
            fixed((1, _GD)),
            fixed((1, 1)),
        ],
        out_specs=pl.BlockSpec((blk,), lambda i: (i,)),
        out_shape=jax.ShapeDtypeStruct((_B,), jnp.float32),
    )(ugT, igT, um, im, w1u, w1i, b1, w2, b2, w3, b3, wfg, wfm, bf)


def kernel(user_indices, item_indices, user_gmf_table, item_gmf_table,
           user_mlp_table, item_mlp_table, W1, b1, W2, b2, W3, b3, Wf, bf):
    ugT = jnp.take(user_gmf_table, user_indices, axis=0).T
    igT = jnp.take(item_gmf_table, item_indices, axis=0).T
    um, im = _sc_mlp(user_indices, item_indices, user_mlp_table, item_mlp_table)
    w1u = W1[:, :_MD].T
    w1i = W1[:, _MD:].T
    wfg = Wf[:, :_GD].T
    wfm = Wf[:, _GD:]
    return _tc_mlp(ugT, igT, um, im, w1u, w1i, b1.reshape(1, _MD),
                   W2.T, b2.reshape(1, 64), W3.T, b3.reshape(1, _GD),
                   wfg, wfm, bf.reshape(1, 1))
